# baseline (speedup 1.0000x reference)
import functools

import jax
import jax.numpy as jnp
from jax import lax
from jax.experimental import pallas as pl
from jax.experimental.pallas import tpu as pltpu
from jax.experimental.pallas import tpu_sc as plsc

B, K, V = 32, 8, 32000
L = 16
NROWS = B * K
NW = 32
ROWS_PER_W = NROWS // NW
NV = V // L
GROUP = 8
NG = NV // GROUP
SUPER = 5
NSG = NG // SUPER
NCAND = K * L


def _insert(Rs, Is, v, idx):
  nR, nI = [], []
  for j in range(K):
    c = v > Rs[j]
    nR.append(jnp.where(c, v, Rs[j]))
    nI.append(jnp.where(c, idx, Is[j]))
    v, idx = jnp.where(c, Rs[j], v), jnp.where(c, Is[j], idx)
  return tuple(nR), tuple(nI)


def _sc_stage(x, nrows, row_off):
  rpw = nrows // NW
  mesh = plsc.VectorSubcoreMesh(core_axis_name="c", subcore_axis_name="s")

  @functools.partial(
      pl.kernel,
      mesh=mesh,
      compiler_params=pltpu.CompilerParams(needs_layout_passes=False),
      out_type=[
          jax.ShapeDtypeStruct((nrows * NCAND,), jnp.float32),
          jax.ShapeDtypeStruct((nrows * NCAND,), jnp.int32),
          jax.ShapeDtypeStruct((nrows * L,), jnp.float32),
      ],
      scratch_types=[
          pltpu.VMEM((V,), jnp.float32),
          pltpu.VMEM((V,), jnp.float32),
          pltpu.VMEM((NG * L,), jnp.float32),
          pltpu.VMEM((rpw * NCAND,), jnp.float32),
          pltpu.VMEM((rpw * NCAND,), jnp.int32),
          pltpu.VMEM((rpw * L,), jnp.float32),
          pltpu.SemaphoreType.DMA,
          pltpu.SemaphoreType.DMA,
      ],
  )
  def sc_k(x_hbm, vals_hbm, ids_hbm, s_hbm, row_a, row_b, gsc, vals_v,
           ids_v, s_v, sem_a, sem_b):
    wid = lax.axis_index("s") * 2 + lax.axis_index("c")
    lane = lax.iota(jnp.int32, 16)
    neg_inf = jnp.float32(-jnp.inf)
    outbase = wid * rpw
    rowbase = row_off + outbase

    bufs = [(row_a, sem_a), (row_b, sem_b)]
    handles = [pltpu.async_copy(x_hbm.at[rowbase], row_a, sem_a)]

    def any_ge(v, th):
      cnt = plsc.all_reduce_population_count(v >= th)
      return cnt[0] > 0

    def make_row(row_v, r, handle, prefetch):
      handle.wait()
      if prefetch is not None:
        nbuf, nsem, nrow = prefetch
        handles.append(pltpu.async_copy(x_hbm.at[nrow], nbuf, nsem))

      def phase_a(h, carry):
        sa, sb, m16 = carry
        for u in range(2):
          g = h * 2 + u
          base = g * (GROUP * L)
          vs = [row_v[pl.ds(base + t * L, L)] for t in range(GROUP)]
          for t in range(0, GROUP, 2):
            sa = sa + jnp.exp(vs[t])
            sb = sb + jnp.exp(vs[t + 1])
          gmax = vs[0]
          for t in range(1, GROUP):
            gmax = jnp.maximum(gmax, vs[t])
          gsc[pl.ds(g * L, L)] = gmax
          m16 = jnp.maximum(m16, gmax)
        return sa, sb, m16

      z = jnp.zeros((L,), jnp.float32)
      ninf16 = jnp.full((L,), neg_inf, jnp.float32)
      sa, sb, m16 = lax.fori_loop(0, NG // 2, phase_a, (z, z, ninf16))

      th = plsc.sort_key_val(m16, lane, descending=True)[0][K - 1]

      def phase_b(sgi, carry):
        gbase = sgi * SUPER
        gms = [gsc[pl.ds((gbase + u) * L, L)] for u in range(SUPER)]
        smax = gms[0]
        for u in range(1, SUPER):
          smax = jnp.maximum(smax, gms[u])

        def super_slow(args):
          def gbody(u, a):
            gm = gsc[pl.ds((gbase + u) * L, L)]

            def gslow(a2):
              def cbody(t, a3):
                ci = (gbase + u) * GROUP + t
                v = row_v[pl.ds(ci * L, L)]

                def ins(a4):
                  return _insert(a4[0], a4[1], v, lane + ci * L)

                return lax.cond(any_ge(v, th), ins, lambda a4: a4, a3)

              return lax.fori_loop(0, GROUP, cbody, a2)

            return lax.cond(any_ge(gm, th), gslow, lambda a2: a2, a)

          return lax.fori_loop(0, SUPER, gbody, args)

        return lax.cond(any_ge(smax, th), super_slow, lambda a: a, carry)

      R0 = tuple(ninf16 for _ in range(K))
      I0 = tuple(jnp.zeros((L,), jnp.int32) for _ in range(K))
      Rs, Is = lax.fori_loop(0, NSG, phase_b, (R0, I0))

      for j in range(K):
        vals_v[pl.ds(r * NCAND + j * L, L)] = Rs[j]
        ids_v[pl.ds(r * NCAND + j * L, L)] = Is[j]
      s_v[pl.ds(r * L, L)] = sa + sb

    for r in range(rpw):
      row_v, _ = bufs[r % 2]
      prefetch = None
      if r + 1 < rpw:
        nbuf, nsem = bufs[(r + 1) % 2]
        prefetch = (nbuf, nsem, rowbase + r + 1)
      make_row(row_v, r, handles[r], prefetch)

    pltpu.sync_copy(vals_v, vals_hbm.at[pl.ds(outbase * NCAND,
                                              rpw * NCAND)])
    pltpu.sync_copy(ids_v, ids_hbm.at[pl.ds(outbase * NCAND,
                                            rpw * NCAND)])
    pltpu.sync_copy(s_v, s_hbm.at[pl.ds(outbase * L, rpw * L)])

  return sc_k(x)


def _sel_compute(cv_ref, ci_ref, s_ref, bs_ref, pa_ref, it_ref,
                 oh_ref, lse_ref, pred_ref, sc_ref):
  bh = cv_ref.shape[0]
  cand_vals = cv_ref[...]
  cand_ids = ci_ref[...]
  s16 = s_ref[...]
  bscores = bs_ref[...]
  pred = pa_ref[...]
  is_term = it_ref[...]

  big = jnp.int32(1 << 30)
  neg_inf = jnp.float32(-jnp.inf)

  lse = jnp.log(jnp.sum(s16, axis=2, keepdims=True))

  work = cand_vals
  tv, ti = [], []
  for _ in range(K):
    m = jnp.max(work, axis=2, keepdims=True)
    hitm = work == m
    idsel = jnp.min(jnp.where(hitm, cand_ids, big), axis=2, keepdims=True)
    tv.append(m)
    ti.append(idsel)
    work = jnp.where(hitm & (cand_ids == idsel), neg_inf, work)
  top_vals = jnp.concatenate(tv, axis=2) - lse
  top_ids = jnp.concatenate(ti, axis=2)

  slot = lax.broadcasted_iota(jnp.int32, (bh, K, K), 2)
  term_scores = jnp.where(slot == 0, 0.0, neg_inf)
  top_vals = jnp.where(is_term > 0, term_scores, top_vals)
  scores = top_vals + bscores

  fidx = (lax.broadcasted_iota(jnp.int32, (bh, K, K), 1) * K
          + slot)
  w = scores
  ns = jnp.zeros((bh, K, 1), jnp.float32)
  sid = jnp.zeros((bh, K, 1), jnp.int32)
  aid = jnp.zeros((bh, K, 1), jnp.int32)
  kslot = lax.broadcasted_iota(jnp.int32, (bh, K, 1), 1)
  for k in range(K):
    m2 = jnp.max(w, axis=2, keepdims=True)
    m = jnp.max(m2, axis=1, keepdims=True)
    hitm = w == m
    s2 = jnp.min(jnp.where(hitm, fidx, big), axis=2, keepdims=True)
    s = jnp.min(s2, axis=1, keepdims=True)
    a2 = jnp.sum(jnp.where(fidx == s, top_ids, 0), axis=2, keepdims=True)
    a = jnp.sum(a2, axis=1, keepdims=True)
    pick = kslot == k
    ns = jnp.where(pick, m, ns)
    sid = jnp.where(pick, s // K, sid)
    aid = jnp.where(pick, a, aid)
    w = jnp.where(fidx == s, neg_inf, w)
  sc_ref[...] = ns

  onehot = (sid == slot).astype(jnp.float32)
  oh_ref[...] = onehot

  lse_g = jnp.zeros((bh, K, 1), jnp.float32)
  pred_g = pred
  for j in range(K):
    pick = sid == j
    lse_g = jnp.where(pick, lse[:, j:j + 1, :], lse_g)
    pred_g = jnp.where(pick, pred[:, j:j + 1, :], pred_g)
  lse_ref[...] = lse_g
  pred_ref[:, :, pl.ds(0, 16)] = pred_g
  pred_ref[:, :, pl.ds(16, 1)] = aid


def _make_tc_body(nprev):
  def _tc_body(*refs):
    (cv_ref, ci_ref, s_ref, bs_ref, pa_ref, it_ref, x_ref) = refs[:7]
    out_ref, pred_ref, sc_ref, oh_sc, lse_sc = refs[7 + nprev:]
    b = pl.program_id(0)

    @pl.when(b == 0)
    def _():
      _sel_compute(cv_ref, ci_ref, s_ref, bs_ref, pa_ref, it_ref,
                   oh_sc, lse_sc, pred_ref, sc_ref)

    onehot = oh_sc[b]
    x = x_ref[0]
    lse_g = lse_sc[b]
    xg = lax.dot_general(onehot, x, (((1,), (0,)), ((), ())),
                         preferred_element_type=jnp.float32)
    out_ref[0] = xg - lse_g

  return _tc_body


def _tc_stage(cand_vals, cand_ids, s16, bscores, predactions, is_term, x,
              boff, prev=None):
  bh = cand_vals.shape[0]
  fix = lambda b: (0, 0, 0)
  bmap = lambda b: (b + boff, 0, 0)
  in_specs = [
      pl.BlockSpec((bh, K, NCAND), fix),
      pl.BlockSpec((bh, K, NCAND), fix),
      pl.BlockSpec((bh, K, L), fix),
      pl.BlockSpec((bh, K, 1), fix),
      pl.BlockSpec((bh, K, 16), fix),
      pl.BlockSpec((bh, K, 1), fix),
      pl.BlockSpec((1, K, V), bmap),
  ]
  args = [cand_vals, cand_ids, s16, bscores, predactions, is_term, x]
  aliases = {}
  if prev is not None:
    in_specs.append(pl.BlockSpec(memory_space=pltpu.MemorySpace.HBM))
    args.append(prev)
    aliases = {7: 0}
  return pl.pallas_call(
      _make_tc_body(0 if prev is None else 1),
      grid=(bh,),
      in_specs=in_specs,
      out_specs=[
          pl.BlockSpec((1, K, V), bmap),
          pl.BlockSpec((bh, K, 17), fix),
          pl.BlockSpec((bh, K, 1), fix),
      ],
      out_shape=[
          jax.ShapeDtypeStruct((B, K, V), jnp.float32),
          jax.ShapeDtypeStruct((bh, K, 17), jnp.int32),
          jax.ShapeDtypeStruct((bh, K, 1), jnp.float32),
      ],
      scratch_shapes=[
          pltpu.VMEM((bh, K, K), jnp.float32),
          pltpu.VMEM((bh, K, 1), jnp.float32),
      ],
      input_output_aliases=aliases,
  )(*args)


def kernel(actionprobs, bscores, predactions, is_term):
  x2 = actionprobs.reshape(NROWS, V)
  bh = B // 2
  nrh = NROWS // 2
  bs3 = bscores.reshape(B, K, 1)
  it3 = is_term.astype(jnp.int32).reshape(B, K, 1)

  cv1, ci1, s1 = _sc_stage(x2, nrh, 0)
  cv2, ci2, s2 = _sc_stage(x2, nrh, nrh)
  out1, pred1, ns1 = _tc_stage(
      cv1.reshape(bh, K, NCAND), ci1.reshape(bh, K, NCAND),
      s1.reshape(bh, K, L), bs3[:bh], predactions[:bh], it3[:bh],
      actionprobs, 0)
  out2, pred2, ns2 = _tc_stage(
      cv2.reshape(bh, K, NCAND), ci2.reshape(bh, K, NCAND),
      s2.reshape(bh, K, L), bs3[bh:], predactions[bh:], it3[bh:],
      actionprobs, bh, prev=out1)
  new_pred = jnp.concatenate([pred1, pred2], axis=0)
  new_scores = jnp.concatenate([ns1, ns2], axis=0).reshape(B, K)
  return (out2, new_pred, new_scores)


# speedup vs baseline: 1.1049x; 1.1049x over previous
import functools

import jax
import jax.numpy as jnp
from jax import lax
from jax.experimental import pallas as pl
from jax.experimental.pallas import tpu as pltpu
from jax.experimental.pallas import tpu_sc as plsc

B, K, V = 32, 8, 32000
L = 16
NROWS = B * K
NW = 32
ROWS_PER_W = NROWS // NW
NV = V // L
GROUP = 8
NG = NV // GROUP
SUPER = 5
NSG = NG // SUPER
NCAND = K * L


def _insert(Rs, Is, v, idx):
  nR, nI = [], []
  for j in range(K):
    c = v > Rs[j]
    nR.append(jnp.where(c, v, Rs[j]))
    nI.append(jnp.where(c, idx, Is[j]))
    v, idx = jnp.where(c, Rs[j], v), jnp.where(c, Is[j], idx)
  return tuple(nR), tuple(nI)


def _sc_stage(x, nrows, row_off):
  rpw = nrows // NW
  mesh = plsc.VectorSubcoreMesh(core_axis_name="c", subcore_axis_name="s")

  @functools.partial(
      pl.kernel,
      mesh=mesh,
      compiler_params=pltpu.CompilerParams(needs_layout_passes=False),
      out_type=[
          jax.ShapeDtypeStruct((nrows * NCAND,), jnp.float32),
          jax.ShapeDtypeStruct((nrows * NCAND,), jnp.int32),
          jax.ShapeDtypeStruct((nrows * L,), jnp.float32),
      ],
      scratch_types=[
          pltpu.VMEM((V,), jnp.float32),
          pltpu.VMEM((V,), jnp.float32),
          pltpu.VMEM((NG * L,), jnp.float32),
          pltpu.VMEM((rpw * NCAND,), jnp.float32),
          pltpu.VMEM((rpw * NCAND,), jnp.int32),
          pltpu.VMEM((rpw * L,), jnp.float32),
          pltpu.SemaphoreType.DMA,
          pltpu.SemaphoreType.DMA,
      ],
  )
  def sc_k(x_hbm, vals_hbm, ids_hbm, s_hbm, row_a, row_b, gsc, vals_v,
           ids_v, s_v, sem_a, sem_b):
    wid = lax.axis_index("s") * 2 + lax.axis_index("c")
    lane = lax.iota(jnp.int32, 16)
    neg_inf = jnp.float32(-jnp.inf)
    outbase = wid * rpw
    rowbase = row_off + outbase

    bufs = [(row_a, sem_a), (row_b, sem_b)]
    handles = [pltpu.async_copy(x_hbm.at[rowbase], row_a, sem_a)]

    def any_ge(v, th):
      cnt = plsc.all_reduce_population_count(v >= th)
      return cnt[0] > 0

    def make_row(row_v, r, handle, prefetch):
      handle.wait()
      if prefetch is not None:
        nbuf, nsem, nrow = prefetch
        handles.append(pltpu.async_copy(x_hbm.at[nrow], nbuf, nsem))

      def phase_a(h, carry):
        sa, sb, m16 = carry
        for u in range(2):
          g = h * 2 + u
          base = g * (GROUP * L)
          vs = [row_v[pl.ds(base + t * L, L)] for t in range(GROUP)]
          for t in range(0, GROUP, 2):
            sa = sa + jnp.exp(vs[t])
            sb = sb + jnp.exp(vs[t + 1])
          gmax = vs[0]
          for t in range(1, GROUP):
            gmax = jnp.maximum(gmax, vs[t])
          gsc[pl.ds(g * L, L)] = gmax
          m16 = jnp.maximum(m16, gmax)
        return sa, sb, m16

      z = jnp.zeros((L,), jnp.float32)
      ninf16 = jnp.full((L,), neg_inf, jnp.float32)
      sa, sb, m16 = lax.fori_loop(0, NG // 2, phase_a, (z, z, ninf16))

      th = plsc.sort_key_val(m16, lane, descending=True)[0][K - 1]

      def phase_b(sgi, carry):
        gbase = sgi * SUPER
        gms = [gsc[pl.ds((gbase + u) * L, L)] for u in range(SUPER)]
        smax = gms[0]
        for u in range(1, SUPER):
          smax = jnp.maximum(smax, gms[u])

        def super_slow(args):
          def gbody(u, a):
            gm = gsc[pl.ds((gbase + u) * L, L)]

            def gslow(a2):
              def cbody(t, a3):
                ci = (gbase + u) * GROUP + t
                v = row_v[pl.ds(ci * L, L)]

                def ins(a4):
                  return _insert(a4[0], a4[1], v, lane + ci * L)

                return lax.cond(any_ge(v, th), ins, lambda a4: a4, a3)

              return lax.fori_loop(0, GROUP, cbody, a2)

            return lax.cond(any_ge(gm, th), gslow, lambda a2: a2, a)

          return lax.fori_loop(0, SUPER, gbody, args)

        return lax.cond(any_ge(smax, th), super_slow, lambda a: a, carry)

      R0 = tuple(ninf16 for _ in range(K))
      I0 = tuple(jnp.zeros((L,), jnp.int32) for _ in range(K))
      Rs, Is = lax.fori_loop(0, NSG, phase_b, (R0, I0))

      for j in range(K):
        vals_v[pl.ds(r * NCAND + j * L, L)] = Rs[j]
        ids_v[pl.ds(r * NCAND + j * L, L)] = Is[j]
      s_v[pl.ds(r * L, L)] = sa + sb

    for r in range(rpw):
      row_v, _ = bufs[r % 2]
      prefetch = None
      if r + 1 < rpw:
        nbuf, nsem = bufs[(r + 1) % 2]
        prefetch = (nbuf, nsem, rowbase + r + 1)
      make_row(row_v, r, handles[r], prefetch)

    pltpu.sync_copy(vals_v, vals_hbm.at[pl.ds(outbase * NCAND,
                                              rpw * NCAND)])
    pltpu.sync_copy(ids_v, ids_hbm.at[pl.ds(outbase * NCAND,
                                            rpw * NCAND)])
    pltpu.sync_copy(s_v, s_hbm.at[pl.ds(outbase * L, rpw * L)])

  return sc_k(x)


def _sel_compute(cv_ref, ci_ref, s_ref, bs_ref, pa_ref, it_ref,
                 oh_ref, lse_ref, pred_ref, sc_ref):
  bh = cv_ref.shape[0]
  cand_vals = cv_ref[...]
  cand_ids = ci_ref[...]
  s16 = s_ref[...]
  bscores = bs_ref[...]
  pred = pa_ref[...]
  is_term = it_ref[...]

  big = jnp.int32(1 << 30)
  neg_inf = jnp.float32(-jnp.inf)

  lse = jnp.log(jnp.sum(s16, axis=2, keepdims=True))

  work = cand_vals
  tv, ti = [], []
  for _ in range(K):
    m = jnp.max(work, axis=2, keepdims=True)
    hitm = work == m
    idsel = jnp.min(jnp.where(hitm, cand_ids, big), axis=2, keepdims=True)
    tv.append(m)
    ti.append(idsel)
    work = jnp.where(hitm & (cand_ids == idsel), neg_inf, work)
  top_vals = jnp.concatenate(tv, axis=2) - lse
  top_ids = jnp.concatenate(ti, axis=2)

  slot = lax.broadcasted_iota(jnp.int32, (bh, K, K), 2)
  term_scores = jnp.where(slot == 0, 0.0, neg_inf)
  top_vals = jnp.where(is_term > 0, term_scores, top_vals)
  scores = top_vals + bscores

  fidx = (lax.broadcasted_iota(jnp.int32, (bh, K, K), 1) * K
          + slot)
  w = scores
  ns = jnp.zeros((bh, K, 1), jnp.float32)
  sid = jnp.zeros((bh, K, 1), jnp.int32)
  aid = jnp.zeros((bh, K, 1), jnp.int32)
  kslot = lax.broadcasted_iota(jnp.int32, (bh, K, 1), 1)
  for k in range(K):
    m2 = jnp.max(w, axis=2, keepdims=True)
    m = jnp.max(m2, axis=1, keepdims=True)
    hitm = w == m
    s2 = jnp.min(jnp.where(hitm, fidx, big), axis=2, keepdims=True)
    s = jnp.min(s2, axis=1, keepdims=True)
    a2 = jnp.sum(jnp.where(fidx == s, top_ids, 0), axis=2, keepdims=True)
    a = jnp.sum(a2, axis=1, keepdims=True)
    pick = kslot == k
    ns = jnp.where(pick, m, ns)
    sid = jnp.where(pick, s // K, sid)
    aid = jnp.where(pick, a, aid)
    w = jnp.where(fidx == s, neg_inf, w)
  sc_ref[...] = ns

  onehot = (sid == slot).astype(jnp.float32)
  oh_ref[...] = onehot

  lse_g = jnp.zeros((bh, K, 1), jnp.float32)
  pred_g = pred
  for j in range(K):
    pick = sid == j
    lse_g = jnp.where(pick, lse[:, j:j + 1, :], lse_g)
    pred_g = jnp.where(pick, pred[:, j:j + 1, :], pred_g)
  lse_ref[...] = lse_g
  pred_ref[:, :, pl.ds(0, 16)] = pred_g
  pred_ref[:, :, pl.ds(16, 1)] = aid


def _make_tc_body(nprev):
  def _tc_body(*refs):
    (cv_ref, ci_ref, s_ref, bs_ref, pa_ref, it_ref, x_ref) = refs[:7]
    out_ref, pred_ref, sc_ref, oh_sc, lse_sc = refs[7 + nprev:]
    b = pl.program_id(0)

    @pl.when(b == 0)
    def _():
      _sel_compute(cv_ref, ci_ref, s_ref, bs_ref, pa_ref, it_ref,
                   oh_sc, lse_sc, pred_ref, sc_ref)

    onehot = oh_sc[b]
    x = x_ref[0]
    lse_g = lse_sc[b]
    xg = lax.dot_general(onehot, x, (((1,), (0,)), ((), ())),
                         preferred_element_type=jnp.float32)
    out_ref[0] = xg - lse_g

  return _tc_body


def _tc_stage(cand_vals, cand_ids, s16, bscores, predactions, is_term, x,
              boff, prev=None):
  bh = cand_vals.shape[0]
  fix = lambda b: (0, 0, 0)
  bmap = lambda b: (b + boff, 0, 0)
  in_specs = [
      pl.BlockSpec((bh, K, NCAND), fix),
      pl.BlockSpec((bh, K, NCAND), fix),
      pl.BlockSpec((bh, K, L), fix),
      pl.BlockSpec((bh, K, 1), fix),
      pl.BlockSpec((bh, K, 16), fix),
      pl.BlockSpec((bh, K, 1), fix),
      pl.BlockSpec((1, K, V), bmap),
  ]
  args = [cand_vals, cand_ids, s16, bscores, predactions, is_term, x]
  aliases = {}
  if prev is not None:
    in_specs.append(pl.BlockSpec(memory_space=pltpu.MemorySpace.HBM))
    args.append(prev)
    aliases = {7: 0}
  return pl.pallas_call(
      _make_tc_body(0 if prev is None else 1),
      grid=(bh,),
      in_specs=in_specs,
      out_specs=[
          pl.BlockSpec((1, K, V), bmap),
          pl.BlockSpec((bh, K, 17), fix),
          pl.BlockSpec((bh, K, 1), fix),
      ],
      out_shape=[
          jax.ShapeDtypeStruct((B, K, V), jnp.float32),
          jax.ShapeDtypeStruct((bh, K, 17), jnp.int32),
          jax.ShapeDtypeStruct((bh, K, 1), jnp.float32),
      ],
      scratch_shapes=[
          pltpu.VMEM((bh, K, K), jnp.float32),
          pltpu.VMEM((bh, K, 1), jnp.float32),
      ],
      input_output_aliases=aliases,
  )(*args)


def kernel(actionprobs, bscores, predactions, is_term):
  x2 = actionprobs.reshape(NROWS, V)
  cand_vals, cand_ids, s16 = _sc_stage(x2, NROWS, 0)
  out_x, new_pred, new_scores = _tc_stage(
      cand_vals.reshape(B, K, NCAND), cand_ids.reshape(B, K, NCAND),
      s16.reshape(B, K, L), bscores.reshape(B, K, 1), predactions,
      is_term.astype(jnp.int32).reshape(B, K, 1), actionprobs, 0)
  return (out_x, new_pred, new_scores.reshape(B, K))
